# Initial kernel scaffold; baseline (speedup 1.0000x reference)
import jax
import jax.numpy as jnp
from jax.experimental import pallas as pl


def kernel(x, edge_index, edge_label_index, W1l, b1l, W1r, ln_g, ln_b, W2l, b2l, W2r, DW1, Db1, DW2, Db2):
    raise NotImplementedError("write your pallas kernel here")


# speedup vs baseline: 4.2549x; 4.2549x over previous
import functools

import jax
import jax.numpy as jnp
from jax import lax
from jax.experimental import pallas as pl
from jax.experimental.pallas import tpu as pltpu
from jax.experimental.pallas import tpu_sc as plsc

N = 10000
E = 320000
L = 100000
D = 128
H = 256
O = 128
DEC_H = 64

NC = 2
NS = 16
NW = NC * NS

_MESH = plsc.VectorSubcoreMesh(core_axis_name="c", subcore_axis_name="s")

EC = E // NW
SEG_CHUNK = 80
SEG_NCHUNK = EC // SEG_CHUNK
NP = 10240
ROWS_PER_TILE = NP // NS


def _make_seg_sum(with_counts: bool):
    out_type = [jax.ShapeDtypeStruct((NC, NP, D), jnp.float32)]
    if with_counts:
        out_type.append(jax.ShapeDtypeStruct((NC, NP), jnp.float32))
    scratch = [
        pltpu.VMEM_SHARED((NP, D), jnp.float32),
        pltpu.VMEM((SEG_CHUNK,), jnp.int32),
        pltpu.VMEM((SEG_CHUNK,), jnp.int32),
        pltpu.VMEM((SEG_CHUNK, D), jnp.float32),
        pltpu.SemaphoreType.DMA,
    ]
    if with_counts:
        scratch += [
            pltpu.VMEM_SHARED((NP,), jnp.float32),
            pltpu.VMEM((SEG_CHUNK,), jnp.float32),
        ]

    def body(table, src, dst, zeros2d, zeros1d, ones, *rest):
        if with_counts:
            agg_out, cnt_out, agg_sp, sidx, didx, rows, sem, cnt_sp, ones_v = rest
        else:
            agg_out, agg_sp, sidx, didx, rows, sem = rest
        cid = lax.axis_index("c")
        sid = lax.axis_index("s")
        wid = sid * NC + cid
        rbase = pl.multiple_of(sid * ROWS_PER_TILE, 8)
        pltpu.sync_copy(zeros2d.at[pl.ds(rbase, ROWS_PER_TILE)],
                        agg_sp.at[pl.ds(rbase, ROWS_PER_TILE)])
        if with_counts:
            pltpu.sync_copy(zeros1d.at[pl.ds(rbase, ROWS_PER_TILE)],
                            cnt_sp.at[pl.ds(rbase, ROWS_PER_TILE)])
            pltpu.sync_copy(ones, ones_v)
        plsc.subcore_barrier()

        def chunk(i, carry):
            base = pl.multiple_of(wid * EC + i * SEG_CHUNK, 8)
            pltpu.sync_copy(src.at[pl.ds(base, SEG_CHUNK)], sidx)
            pltpu.sync_copy(dst.at[pl.ds(base, SEG_CHUNK)], didx)
            pltpu.async_copy(table.at[sidx], rows, sem).wait()
            pltpu.sync_copy(rows, agg_sp.at[didx], add=True)
            if with_counts:
                pltpu.sync_copy(ones_v, cnt_sp.at[didx], add=True)
            return carry

        lax.fori_loop(0, SEG_NCHUNK, chunk, 0)
        plsc.subcore_barrier()
        pltpu.sync_copy(agg_sp.at[pl.ds(rbase, ROWS_PER_TILE)],
                        agg_out.at[cid, pl.ds(rbase, ROWS_PER_TILE)])
        if with_counts:
            pltpu.sync_copy(cnt_sp.at[pl.ds(rbase, ROWS_PER_TILE)],
                            cnt_out.at[cid, pl.ds(rbase, ROWS_PER_TILE)])

    return pl.kernel(body, out_type=tuple(out_type), mesh=_MESH,
                     scratch_types=scratch)


_seg_sum_counts = _make_seg_sum(True)
_seg_sum = _make_seg_sum(False)

BM = 1000


def _enc1_body(x_ref, p_ref, cnt_ref, w1lt_ref, w1rt_ref, b1l_ref, g_ref,
               b_ref, h0_ref, h1_ref, inv_ref):
    cnt = cnt_ref[:, 0] + cnt_ref[:, 1]
    inv = 1.0 / jnp.maximum(cnt, 1.0)
    mean = (p_ref[0] + p_ref[1]) * inv[:, None]
    pre = (jnp.dot(mean, w1lt_ref[...], preferred_element_type=jnp.float32)
           + jnp.dot(x_ref[...], w1rt_ref[...], preferred_element_type=jnp.float32)
           + b1l_ref[...])
    mu = jnp.mean(pre, axis=-1, keepdims=True)
    var = jnp.mean((pre - mu) ** 2, axis=-1, keepdims=True)
    hh = (pre - mu) * lax.rsqrt(var + 1e-5) * g_ref[...] + b_ref[...]
    hh = jnp.maximum(hh, 0.0)
    h0_ref[...] = hh[:, :O]
    h1_ref[...] = hh[:, O:]
    inv_ref[...] = inv[:, None]


def _encoder1(x, parts, cnt_parts, w1lt, w1rt, b1l, ln_g, ln_b):
    grid = (N // BM,)
    return pl.pallas_call(
        _enc1_body,
        grid=grid,
        in_specs=[
            pl.BlockSpec((BM, D), lambda i: (i, 0)),
            pl.BlockSpec((NC, BM, D), lambda i: (0, i, 0)),
            pl.BlockSpec((BM, NC), lambda i: (i, 0)),
            pl.BlockSpec((D, H), lambda i: (0, 0)),
            pl.BlockSpec((D, H), lambda i: (0, 0)),
            pl.BlockSpec((1, H), lambda i: (0, 0)),
            pl.BlockSpec((1, H), lambda i: (0, 0)),
            pl.BlockSpec((1, H), lambda i: (0, 0)),
        ],
        out_specs=[
            pl.BlockSpec((BM, O), lambda i: (i, 0)),
            pl.BlockSpec((BM, O), lambda i: (i, 0)),
            pl.BlockSpec((BM, 1), lambda i: (i, 0)),
        ],
        out_shape=[
            jax.ShapeDtypeStruct((N, O), jnp.float32),
            jax.ShapeDtypeStruct((N, O), jnp.float32),
            jax.ShapeDtypeStruct((N, 1), jnp.float32),
        ],
    )(x, parts, cnt_parts, w1lt, w1rt, b1l, ln_g, ln_b)


def _enc2_body(h0_ref, h1_ref, a0_ref, a1_ref, inv_ref, w2lt_ref, w2rt_ref,
               b2l_ref, dw1t_ref, bpq_ref, p_ref, q_ref):
    inv = inv_ref[...]
    m0 = (a0_ref[0] + a0_ref[1]) * inv
    m1 = (a1_ref[0] + a1_ref[1]) * inv
    w2lt = w2lt_ref[...]
    w2rt = w2rt_ref[...]
    z = (jnp.dot(m0, w2lt[:O], preferred_element_type=jnp.float32)
         + jnp.dot(m1, w2lt[O:], preferred_element_type=jnp.float32)
         + jnp.dot(h0_ref[...], w2rt[:O], preferred_element_type=jnp.float32)
         + jnp.dot(h1_ref[...], w2rt[O:], preferred_element_type=jnp.float32)
         + b2l_ref[...])
    nrm = jnp.maximum(jnp.sqrt(jnp.sum(z * z, axis=-1, keepdims=True)), 1e-12)
    z = z / nrm
    pq = jnp.dot(z, dw1t_ref[...], preferred_element_type=jnp.float32) + bpq_ref[...]
    p_ref[...] = pq[:, :DEC_H]
    q_ref[...] = pq[:, DEC_H:]


def _encoder2(h0, h1, a0, a1, inv_cnt, w2lt, w2rt, b2l, dw1t, bpq):
    grid = (N // BM,)
    return pl.pallas_call(
        _enc2_body,
        grid=grid,
        in_specs=[
            pl.BlockSpec((BM, O), lambda i: (i, 0)),
            pl.BlockSpec((BM, O), lambda i: (i, 0)),
            pl.BlockSpec((NC, BM, O), lambda i: (0, i, 0)),
            pl.BlockSpec((NC, BM, O), lambda i: (0, i, 0)),
            pl.BlockSpec((BM, 1), lambda i: (i, 0)),
            pl.BlockSpec((H, O), lambda i: (0, 0)),
            pl.BlockSpec((H, O), lambda i: (0, 0)),
            pl.BlockSpec((1, O), lambda i: (0, 0)),
            pl.BlockSpec((O, 2 * DEC_H), lambda i: (0, 0)),
            pl.BlockSpec((1, 2 * DEC_H), lambda i: (0, 0)),
        ],
        out_specs=[
            pl.BlockSpec((BM, DEC_H), lambda i: (i, 0)),
            pl.BlockSpec((BM, DEC_H), lambda i: (i, 0)),
        ],
        out_shape=[
            jax.ShapeDtypeStruct((N, DEC_H), jnp.float32),
            jax.ShapeDtypeStruct((N, DEC_H), jnp.float32),
        ],
    )(h0, h1, a0, a1, inv_cnt, w2lt, w2rt, b2l, dw1t, bpq)


LP = 102400
LW = LP // NW
DEC_CHUNK = 128
DEC_NCHUNK = LW // DEC_CHUNK


def _dec_body(p_hbm, q_hbm, sidx_hbm, didx_hbm, w_hbm, b2_hbm, out_hbm,
              sidx, didx, prows, qrows, wbuf, b2buf, tmp, outbuf, sem):
    cid = lax.axis_index("c")
    sid = lax.axis_index("s")
    wid = sid * NC + cid
    pltpu.sync_copy(w_hbm, wbuf)
    pltpu.sync_copy(b2_hbm, b2buf)
    wregs = [wbuf[pl.ds(k * 16, 16)] for k in range(DEC_H // 16)]
    b2v = b2buf[...]
    iota16x = lax.iota(jnp.int32, 16) * 16

    def chunk(i, carry):
        base = pl.multiple_of(wid * LW + i * DEC_CHUNK, 8)
        pltpu.sync_copy(sidx_hbm.at[pl.ds(base, DEC_CHUNK)], sidx)
        pltpu.sync_copy(didx_hbm.at[pl.ds(base, DEC_CHUNK)], didx)
        c1 = pltpu.async_copy(p_hbm.at[sidx], prows, sem)
        c2 = pltpu.async_copy(q_hbm.at[didx], qrows, sem)
        c1.wait()
        c2.wait()
        for g in range(DEC_CHUNK // 16):
            for e in range(16):
                row = g * 16 + e
                acc = None
                for k in range(DEC_H // 16):
                    pv = prows[row, pl.ds(k * 16, 16)]
                    qv = qrows[row, pl.ds(k * 16, 16)]
                    t = jnp.maximum(pv + qv, 0.0) * wregs[k]
                    acc = t if acc is None else acc + t
                tmp[pl.ds(e * 16, 16)] = acc
            s = b2v
            for j in range(16):
                s = s + plsc.load_gather(tmp, [iota16x + j])
            outbuf[pl.ds(g * 16, 16)] = s
        pltpu.sync_copy(outbuf, out_hbm.at[pl.ds(base, DEC_CHUNK)])
        return carry

    lax.fori_loop(0, DEC_NCHUNK, chunk, 0)


_decoder = pl.kernel(
    _dec_body,
    out_type=jax.ShapeDtypeStruct((LP,), jnp.float32),
    mesh=_MESH,
    compiler_params=pltpu.CompilerParams(needs_layout_passes=False, use_tc_tiling_on_sc=False),
    scratch_types=[
        pltpu.VMEM((DEC_CHUNK,), jnp.int32),
        pltpu.VMEM((DEC_CHUNK,), jnp.int32),
        pltpu.VMEM((DEC_CHUNK, DEC_H), jnp.float32),
        pltpu.VMEM((DEC_CHUNK, DEC_H), jnp.float32),
        pltpu.VMEM((DEC_H,), jnp.float32),
        pltpu.VMEM((16,), jnp.float32),
        pltpu.VMEM((256,), jnp.float32),
        pltpu.VMEM((DEC_CHUNK,), jnp.float32),
        pltpu.SemaphoreType.DMA,
    ],
)


def kernel(x, edge_index, edge_label_index, W1l, b1l, W1r, ln_g, ln_b,
           W2l, b2l, W2r, DW1, Db1, DW2, Db2):
    src = edge_index[0]
    dst = edge_index[1]
    zeros2d = jnp.zeros((NP, D), jnp.float32)
    zeros1d = jnp.zeros((NP,), jnp.float32)
    ones = jnp.ones((SEG_CHUNK,), jnp.float32)

    parts1, cnt_parts = _seg_sum_counts(x, src, dst, zeros2d, zeros1d, ones)

    h0, h1, inv_cnt = _encoder1(
        x, parts1, cnt_parts.T, W1l.T, W1r.T,
        b1l.reshape(1, H), ln_g.reshape(1, H), ln_b.reshape(1, H))

    (a0,) = _seg_sum(h0, src, dst, zeros2d, zeros1d, ones)
    (a1,) = _seg_sum(h1, src, dst, zeros2d, zeros1d, ones)

    dw1t = jnp.concatenate([DW1[:, :O].T, DW1[:, O:].T], axis=1)
    bpq = jnp.concatenate([Db1, jnp.zeros((DEC_H,), jnp.float32)]).reshape(1, 2 * DEC_H)
    p_tab, q_tab = _encoder2(h0, h1, a0, a1, inv_cnt, W2l.T, W2r.T,
                             b2l.reshape(1, O), dw1t, bpq)

    s_idx = jnp.pad(edge_label_index[0], (0, LP - L))
    d_idx = jnp.pad(edge_label_index[1], (0, LP - L))
    w64 = DW2.reshape(DEC_H)
    b2_16 = jnp.broadcast_to(Db2, (16,))
    out = _decoder(p_tab, q_tab, s_idx, d_idx, w64, b2_16)
    return out[:L]


# speedup vs baseline: 6.6064x; 1.5526x over previous
import jax
import jax.numpy as jnp
from jax import lax
from jax.experimental import pallas as pl
from jax.experimental.pallas import tpu as pltpu
from jax.experimental.pallas import tpu_sc as plsc

N = 10000
E = 320000
L = 100000
D = 128
H = 256
O = 128
DEC_H = 64

NC = 2
NS = 16
NW = NC * NS

_MESH = plsc.VectorSubcoreMesh(core_axis_name="c", subcore_axis_name="s")
_SC_PARAMS = pltpu.CompilerParams(needs_layout_passes=False,
                                  use_tc_tiling_on_sc=False)

EC = E // NW
SEG_CHUNK = 80
SEG_NCHUNK = EC // SEG_CHUNK
NP = 10240
ROWS_PER_TILE = NP // NS


def _make_seg_sum(with_counts: bool):
    out_type = [jax.ShapeDtypeStruct((NC, NP, D), jnp.float32)]
    if with_counts:
        out_type.append(jax.ShapeDtypeStruct((NC, NP), jnp.float32))
    scratch = [
        pltpu.VMEM_SHARED((NP, D), jnp.float32),
        pltpu.VMEM((SEG_CHUNK,), jnp.int32),
        pltpu.VMEM((SEG_CHUNK,), jnp.int32),
        pltpu.VMEM((SEG_CHUNK,), jnp.int32),
        pltpu.VMEM((SEG_CHUNK,), jnp.int32),
        pltpu.VMEM((SEG_CHUNK, D), jnp.float32),
        pltpu.VMEM((SEG_CHUNK, D), jnp.float32),
        pltpu.SemaphoreType.DMA,
        pltpu.SemaphoreType.DMA,
    ]
    if with_counts:
        scratch += [
            pltpu.VMEM_SHARED((NP,), jnp.float32),
            pltpu.VMEM((SEG_CHUNK,), jnp.float32),
        ]

    def body(table, src, dst, zeros2d, zeros1d, ones, *rest):
        if with_counts:
            (agg_out, cnt_out, agg_sp, sidx0, didx0, sidx1, didx1,
             rows0, rows1, sem0, sem1, cnt_sp, ones_v) = rest
        else:
            (agg_out, agg_sp, sidx0, didx0, sidx1, didx1,
             rows0, rows1, sem0, sem1) = rest
        cid = lax.axis_index("c")
        sid = lax.axis_index("s")
        wid = sid * NC + cid
        rbase = pl.multiple_of(sid * ROWS_PER_TILE, 8)
        pltpu.sync_copy(zeros2d.at[pl.ds(rbase, ROWS_PER_TILE)],
                        agg_sp.at[pl.ds(rbase, ROWS_PER_TILE)])
        if with_counts:
            pltpu.sync_copy(zeros1d.at[pl.ds(rbase, ROWS_PER_TILE)],
                            cnt_sp.at[pl.ds(rbase, ROWS_PER_TILE)])
            pltpu.sync_copy(ones, ones_v)
        plsc.subcore_barrier()

        ebase = wid * EC

        def load_idx(c, sbuf, dbuf):
            base = pl.multiple_of(ebase + c * SEG_CHUNK, 8)
            pltpu.sync_copy(src.at[pl.ds(base, SEG_CHUNK)], sbuf)
            pltpu.sync_copy(dst.at[pl.ds(base, SEG_CHUNK)], dbuf)

        def scatter(dbuf, rbuf):
            pltpu.sync_copy(rbuf, agg_sp.at[dbuf], add=True)
            if with_counts:
                pltpu.sync_copy(ones_v, cnt_sp.at[dbuf], add=True)

        tmax = SEG_NCHUNK - 1
        load_idx(0, sidx0, didx0)
        pltpu.async_copy(table.at[sidx0], rows0, sem0)

        def pair(i, carry):
            a = 2 * i
            load_idx(jnp.minimum(a + 1, tmax), sidx1, didx1)
            pltpu.async_copy(table.at[sidx1], rows1, sem1)
            pltpu.make_async_copy(table.at[sidx0], rows0, sem0).wait()
            scatter(didx0, rows0)
            load_idx(jnp.minimum(a + 2, tmax), sidx0, didx0)
            pltpu.async_copy(table.at[sidx0], rows0, sem0)
            pltpu.make_async_copy(table.at[sidx1], rows1, sem1).wait()

            @pl.when(a + 1 <= tmax)
            def _():
                scatter(didx1, rows1)

            return carry

        lax.fori_loop(0, (SEG_NCHUNK + 1) // 2, pair, 0)
        pltpu.make_async_copy(table.at[sidx0], rows0, sem0).wait()

        plsc.subcore_barrier()
        pltpu.sync_copy(agg_sp.at[pl.ds(rbase, ROWS_PER_TILE)],
                        agg_out.at[cid, pl.ds(rbase, ROWS_PER_TILE)])
        if with_counts:
            pltpu.sync_copy(cnt_sp.at[pl.ds(rbase, ROWS_PER_TILE)],
                            cnt_out.at[cid, pl.ds(rbase, ROWS_PER_TILE)])

    return pl.kernel(body, out_type=tuple(out_type), mesh=_MESH,
                     compiler_params=_SC_PARAMS, scratch_types=scratch)


_seg_sum_counts = _make_seg_sum(True)
_seg_sum = _make_seg_sum(False)

BM = 1000


def _enc1_body(x_ref, p_ref, cnt_ref, w1lt_ref, w1rt_ref, b1l_ref, g_ref,
               b_ref, h0_ref, h1_ref, inv_ref):
    cnt = cnt_ref[:, 0] + cnt_ref[:, 1]
    inv = 1.0 / jnp.maximum(cnt, 1.0)
    mean = (p_ref[0] + p_ref[1]) * inv[:, None]
    pre = (jnp.dot(mean, w1lt_ref[...], preferred_element_type=jnp.float32)
           + jnp.dot(x_ref[...], w1rt_ref[...], preferred_element_type=jnp.float32)
           + b1l_ref[...])
    mu = jnp.mean(pre, axis=-1, keepdims=True)
    var = jnp.mean((pre - mu) ** 2, axis=-1, keepdims=True)
    hh = (pre - mu) * lax.rsqrt(var + 1e-5) * g_ref[...] + b_ref[...]
    hh = jnp.maximum(hh, 0.0)
    h0_ref[...] = hh[:, :O]
    h1_ref[...] = hh[:, O:]
    inv_ref[...] = inv[:, None]


def _encoder1(x, parts, cnt_parts, w1lt, w1rt, b1l, ln_g, ln_b):
    grid = (N // BM,)
    return pl.pallas_call(
        _enc1_body,
        grid=grid,
        in_specs=[
            pl.BlockSpec((BM, D), lambda i: (i, 0)),
            pl.BlockSpec((NC, BM, D), lambda i: (0, i, 0)),
            pl.BlockSpec((BM, NC), lambda i: (i, 0)),
            pl.BlockSpec((D, H), lambda i: (0, 0)),
            pl.BlockSpec((D, H), lambda i: (0, 0)),
            pl.BlockSpec((1, H), lambda i: (0, 0)),
            pl.BlockSpec((1, H), lambda i: (0, 0)),
            pl.BlockSpec((1, H), lambda i: (0, 0)),
        ],
        out_specs=[
            pl.BlockSpec((BM, O), lambda i: (i, 0)),
            pl.BlockSpec((BM, O), lambda i: (i, 0)),
            pl.BlockSpec((BM, 1), lambda i: (i, 0)),
        ],
        out_shape=[
            jax.ShapeDtypeStruct((N, O), jnp.float32),
            jax.ShapeDtypeStruct((N, O), jnp.float32),
            jax.ShapeDtypeStruct((N, 1), jnp.float32),
        ],
    )(x, parts, cnt_parts, w1lt, w1rt, b1l, ln_g, ln_b)


def _enc2_body(h0_ref, h1_ref, a0_ref, a1_ref, inv_ref, w2lt_ref, w2rt_ref,
               b2l_ref, dw1t_ref, bpq_ref, p_ref, q_ref):
    inv = inv_ref[...]
    m0 = (a0_ref[0] + a0_ref[1]) * inv
    m1 = (a1_ref[0] + a1_ref[1]) * inv
    w2lt = w2lt_ref[...]
    w2rt = w2rt_ref[...]
    z = (jnp.dot(m0, w2lt[:O], preferred_element_type=jnp.float32)
         + jnp.dot(m1, w2lt[O:], preferred_element_type=jnp.float32)
         + jnp.dot(h0_ref[...], w2rt[:O], preferred_element_type=jnp.float32)
         + jnp.dot(h1_ref[...], w2rt[O:], preferred_element_type=jnp.float32)
         + b2l_ref[...])
    nrm = jnp.maximum(jnp.sqrt(jnp.sum(z * z, axis=-1, keepdims=True)), 1e-12)
    z = z / nrm
    pq = jnp.dot(z, dw1t_ref[...], preferred_element_type=jnp.float32) + bpq_ref[...]
    p_ref[...] = pq[:, :DEC_H]
    q_ref[...] = pq[:, DEC_H:]


def _encoder2(h0, h1, a0, a1, inv_cnt, w2lt, w2rt, b2l, dw1t, bpq):
    grid = (N // BM,)
    return pl.pallas_call(
        _enc2_body,
        grid=grid,
        in_specs=[
            pl.BlockSpec((BM, O), lambda i: (i, 0)),
            pl.BlockSpec((BM, O), lambda i: (i, 0)),
            pl.BlockSpec((NC, BM, O), lambda i: (0, i, 0)),
            pl.BlockSpec((NC, BM, O), lambda i: (0, i, 0)),
            pl.BlockSpec((BM, 1), lambda i: (i, 0)),
            pl.BlockSpec((H, O), lambda i: (0, 0)),
            pl.BlockSpec((H, O), lambda i: (0, 0)),
            pl.BlockSpec((1, O), lambda i: (0, 0)),
            pl.BlockSpec((O, 2 * DEC_H), lambda i: (0, 0)),
            pl.BlockSpec((1, 2 * DEC_H), lambda i: (0, 0)),
        ],
        out_specs=[
            pl.BlockSpec((BM, DEC_H), lambda i: (i, 0)),
            pl.BlockSpec((BM, DEC_H), lambda i: (i, 0)),
        ],
        out_shape=[
            jax.ShapeDtypeStruct((N, DEC_H), jnp.float32),
            jax.ShapeDtypeStruct((N, DEC_H), jnp.float32),
        ],
    )(h0, h1, a0, a1, inv_cnt, w2lt, w2rt, b2l, dw1t, bpq)


LP = 102400
LW = LP // NW
DEC_CHUNK = 64
DEC_NCHUNK = LW // DEC_CHUNK


def _dec_body(p_hbm, q_hbm, sidx_hbm, didx_hbm, w_hbm, b2_hbm, out_hbm,
              sidx0, didx0, sidx1, didx1, prows0, qrows0, prows1, qrows1,
              wbuf, b2buf, tmp, outbuf, sem0, sem1):
    cid = lax.axis_index("c")
    sid = lax.axis_index("s")
    wid = sid * NC + cid
    pltpu.sync_copy(w_hbm, wbuf)
    pltpu.sync_copy(b2_hbm, b2buf)
    wregs = [wbuf[pl.ds(k * 16, 16)] for k in range(DEC_H // 16)]
    b2v = b2buf[...]
    iota16x = lax.iota(jnp.int32, 16) * 16
    lbase = wid * LW

    def load_idx(c, sbuf, dbuf):
        base = pl.multiple_of(lbase + c * DEC_CHUNK, 8)
        pltpu.sync_copy(sidx_hbm.at[pl.ds(base, DEC_CHUNK)], sbuf)
        pltpu.sync_copy(didx_hbm.at[pl.ds(base, DEC_CHUNK)], dbuf)

    def start_gather(sbuf, dbuf, prows, qrows, sem):
        pltpu.async_copy(p_hbm.at[sbuf], prows, sem)
        pltpu.async_copy(q_hbm.at[dbuf], qrows, sem)

    def wait_gather(sbuf, dbuf, prows, qrows, sem):
        pltpu.make_async_copy(p_hbm.at[sbuf], prows, sem).wait()
        pltpu.make_async_copy(q_hbm.at[dbuf], qrows, sem).wait()

    def compute(c, prows, qrows):
        for g in range(DEC_CHUNK // 16):
            for e in range(16):
                row = g * 16 + e
                acc = None
                for k in range(DEC_H // 16):
                    pv = prows[row, pl.ds(k * 16, 16)]
                    qv = qrows[row, pl.ds(k * 16, 16)]
                    t = jnp.maximum(pv + qv, 0.0) * wregs[k]
                    acc = t if acc is None else acc + t
                tmp[pl.ds(e * 16, 16)] = acc
            s = b2v
            for j in range(16):
                s = s + plsc.load_gather(tmp, [iota16x + j])
            outbuf[pl.ds(g * 16, 16)] = s
        base = pl.multiple_of(lbase + c * DEC_CHUNK, 8)
        pltpu.sync_copy(outbuf, out_hbm.at[pl.ds(base, DEC_CHUNK)])

    tmax = DEC_NCHUNK - 1
    load_idx(0, sidx0, didx0)
    start_gather(sidx0, didx0, prows0, qrows0, sem0)

    def pair(i, carry):
        a = 2 * i
        load_idx(jnp.minimum(a + 1, tmax), sidx1, didx1)
        start_gather(sidx1, didx1, prows1, qrows1, sem1)
        wait_gather(sidx0, didx0, prows0, qrows0, sem0)
        compute(a, prows0, qrows0)
        load_idx(jnp.minimum(a + 2, tmax), sidx0, didx0)
        start_gather(sidx0, didx0, prows0, qrows0, sem0)
        wait_gather(sidx1, didx1, prows1, qrows1, sem1)

        @pl.when(a + 1 <= tmax)
        def _():
            compute(a + 1, prows1, qrows1)

        return carry

    lax.fori_loop(0, DEC_NCHUNK // 2, pair, 0)
    wait_gather(sidx0, didx0, prows0, qrows0, sem0)


_decoder = pl.kernel(
    _dec_body,
    out_type=jax.ShapeDtypeStruct((LP,), jnp.float32),
    mesh=_MESH,
    compiler_params=_SC_PARAMS,
    scratch_types=[
        pltpu.VMEM((DEC_CHUNK,), jnp.int32),
        pltpu.VMEM((DEC_CHUNK,), jnp.int32),
        pltpu.VMEM((DEC_CHUNK,), jnp.int32),
        pltpu.VMEM((DEC_CHUNK,), jnp.int32),
        pltpu.VMEM((DEC_CHUNK, DEC_H), jnp.float32),
        pltpu.VMEM((DEC_CHUNK, DEC_H), jnp.float32),
        pltpu.VMEM((DEC_CHUNK, DEC_H), jnp.float32),
        pltpu.VMEM((DEC_CHUNK, DEC_H), jnp.float32),
        pltpu.VMEM((DEC_H,), jnp.float32),
        pltpu.VMEM((16,), jnp.float32),
        pltpu.VMEM((256,), jnp.float32),
        pltpu.VMEM((DEC_CHUNK,), jnp.float32),
        pltpu.SemaphoreType.DMA,
        pltpu.SemaphoreType.DMA,
    ],
)


def kernel(x, edge_index, edge_label_index, W1l, b1l, W1r, ln_g, ln_b,
           W2l, b2l, W2r, DW1, Db1, DW2, Db2):
    src = edge_index[0]
    dst = edge_index[1]
    zeros2d = jnp.zeros((NP, D), jnp.float32)
    zeros1d = jnp.zeros((NP,), jnp.float32)
    ones = jnp.ones((SEG_CHUNK,), jnp.float32)

    parts1, cnt_parts = _seg_sum_counts(x, src, dst, zeros2d, zeros1d, ones)

    h0, h1, inv_cnt = _encoder1(
        x, parts1, cnt_parts.T, W1l.T, W1r.T,
        b1l.reshape(1, H), ln_g.reshape(1, H), ln_b.reshape(1, H))

    (a0,) = _seg_sum(h0, src, dst, zeros2d, zeros1d, ones)
    (a1,) = _seg_sum(h1, src, dst, zeros2d, zeros1d, ones)

    dw1t = jnp.concatenate([DW1[:, :O].T, DW1[:, O:].T], axis=1)
    bpq = jnp.concatenate([Db1, jnp.zeros((DEC_H,), jnp.float32)]).reshape(1, 2 * DEC_H)
    p_tab, q_tab = _encoder2(h0, h1, a0, a1, inv_cnt, W2l.T, W2r.T,
                             b2l.reshape(1, O), dw1t, bpq)

    s_idx = jnp.pad(edge_label_index[0], (0, LP - L))
    d_idx = jnp.pad(edge_label_index[1], (0, LP - L))
    w64 = DW2.reshape(DEC_H)
    b2_16 = jnp.broadcast_to(Db2, (16,))
    out = _decoder(p_tab, q_tab, s_idx, d_idx, w64, b2_16)
    return out[:L]
